# baseline (speedup 1.0000x reference)
import functools

import jax
import jax.numpy as jnp
from jax import lax
from jax.experimental import pallas as pl
from jax.experimental.pallas import tpu as pltpu
from jax.experimental.pallas import tpu_sc as plsc

D = 32
H = 256
NSUB = 16
SHW = NSUB * H

N_V = 100000
N_E = 500000
N_T = 500000

NW = 32
NC = 2

NVP = 102400
NEP = 524288
NTP = 524288

SB = 1024
NSV = NVP // SB
NSE = NEP // SB
NST = NTP // SB

VBLK = 2048
NGRID = NVP // VBLK


def _maxsq_body(x_ref, y_ref, z_ref, o_ref):
    i = pl.program_id(0)
    x = x_ref[...]
    y = y_ref[...]
    z = z_ref[...]
    m = jnp.max(x * x + y * y + z * z)

    @pl.when(i == 0)
    def _():
        o_ref[0, 0] = m

    @pl.when(i > 0)
    def _():
        o_ref[0, 0] = jnp.maximum(o_ref[0, 0], m)


def _quant_body(ms_ref, x_ref, y_ref, z_ref, d_ref, o_ref):
    m = jnp.sqrt(ms_ref[0, 0])
    dm = d_ref[...]
    h = (dm[:, 0:1] * x_ref[...]
         + dm[:, 1:2] * y_ref[...]
         + dm[:, 2:3] * z_ref[...])
    idx = jnp.ceil((jnp.float32(H - 1) * (m + h)) / (jnp.float32(2.0) * m))
    idx = jnp.clip(idx, 0.0, jnp.float32(H - 1))
    o_ref[...] = idx.astype(jnp.int32)


def _fin_body(h_ref, o_ref):
    x = h_ref[...]
    j = lax.broadcasted_iota(jnp.int32, (D, NW * NSUB), 1)
    dd = lax.broadcasted_iota(jnp.int32, (D, NW * NSUB), 0)
    sel = ((j // NSUB) == dd).astype(jnp.float32)
    acc = jnp.dot(sel, x, preferred_element_type=jnp.float32)
    s_i = lax.broadcasted_iota(jnp.int32, (H, H), 0)
    t_i = lax.broadcasted_iota(jnp.int32, (H, H), 1)
    tri = (s_i <= t_i).astype(jnp.float32)
    o_ref[...] = jnp.dot(acc, tri, preferred_element_type=jnp.float32)


def _sc_hist_body(tbl, vw, ev0, ev1, ew, tv0, tv1, tv2, tw, out,
                  binv, subh, ids, wbuf, sem):
    wid = lax.axis_index("s") * NC + lax.axis_index("c")

    zeros16 = jnp.zeros((16,), jnp.float32)

    def zi(i, _):
        subh[pl.ds(i * 16, 16)] = zeros16
        return 0

    lax.fori_loop(0, SHW // 16, zi, 0)

    pltpu.sync_copy(tbl.at[pl.ds(wid * NVP, NVP)], binv)

    laneoff = lax.iota(jnp.int32, 16) * H

    def vsb(k, carry):
        pltpu.sync_copy(vw.at[pl.ds(k * SB, SB)], wbuf.at[0])

        def vbody(g, c2):
            b16 = binv[pl.ds(k * SB + g * 16, 16)]
            w16 = wbuf[0, pl.ds(g * 16, 16)]
            plsc.addupdate_scatter(subh, [b16 + laneoff], w16)
            return c2

        return lax.fori_loop(0, SB // 16, vbody, carry, unroll=8)

    lax.fori_loop(0, NSV, vsb, 0)

    def simplex_pass(cols, w_hbm, n_super, negate):
        rows_per = len(cols)

        def stage(k, p):
            for r in range(rows_per):
                pltpu.async_copy(cols[r].at[pl.ds(k * SB, SB)],
                                 ids.at[p, pl.ds(r * SB, SB)],
                                 sem.at[p])
            pltpu.async_copy(w_hbm.at[pl.ds(k * SB, SB)], wbuf.at[p],
                             sem.at[p])

        def wait(p):
            pltpu.make_async_copy(
                cols[0].at[pl.ds(0, rows_per * SB)],
                ids.at[p, pl.ds(0, rows_per * SB)],
                sem.at[p]).wait()
            pltpu.make_async_copy(
                w_hbm.at[pl.ds(0, SB)], wbuf.at[p], sem.at[p]).wait()

        def compute(p):
            def body(g, c2):
                i0 = ids[p, pl.ds(g * 16, 16)]
                i1 = ids[p, pl.ds(SB + g * 16, 16)]
                b = jnp.maximum(plsc.load_gather(binv, [i0]),
                                plsc.load_gather(binv, [i1]))
                if rows_per == 3:
                    i2 = ids[p, pl.ds(2 * SB + g * 16, 16)]
                    b = jnp.maximum(b, plsc.load_gather(binv, [i2]))
                w16 = wbuf[p, pl.ds(g * 16, 16)]
                if negate:
                    w16 = -w16
                plsc.addupdate_scatter(subh, [b + laneoff], w16)
                return c2

            lax.fori_loop(0, SB // 16, body, 0, unroll=8)

        stage(0, 0)

        def pair(j, carry):
            stage(2 * j + 1, 1)
            wait(0)
            compute(0)

            @pl.when(j + 1 < n_super // 2)
            def _():
                stage(2 * j + 2, 0)

            wait(1)
            compute(1)
            return carry

        lax.fori_loop(0, n_super // 2, pair, 0)

    simplex_pass([ev0, ev1], ew, NSE, False)
    simplex_pass([tv0, tv1, tv2], tw, NST, True)

    pltpu.sync_copy(subh, out.at[pl.ds(wid * SHW, SHW)])


def _make_sc_hist():
    mesh = plsc.VectorSubcoreMesh(core_axis_name="c", subcore_axis_name="s")
    return functools.partial(
        pl.kernel,
        mesh=mesh,
        compiler_params=pltpu.CompilerParams(
            needs_layout_passes=False, use_tc_tiling_on_sc=False),
        out_type=jax.ShapeDtypeStruct((NW * SHW,), jnp.float32),
        scratch_types=[
            pltpu.VMEM((NVP,), jnp.int32),
            pltpu.VMEM((SHW,), jnp.float32),
            pltpu.VMEM((2, 3 * SB), jnp.int32),
            pltpu.VMEM((2, SB), jnp.float32),
            pltpu.SemaphoreType.DMA((2,)),
        ],
    )(_sc_hist_body)


_sc_hist = _make_sc_hist()


def kernel(v_coords, v_weights, edge_verts, edge_weights, tri_verts,
           tri_weights, dirs):
    cx = jnp.pad(v_coords[:, 0], (0, NVP - N_V)).reshape(1, NVP)
    cy = jnp.pad(v_coords[:, 1], (0, NVP - N_V)).reshape(1, NVP)
    cz = jnp.pad(v_coords[:, 2], (0, NVP - N_V)).reshape(1, NVP)
    vwp = jnp.pad(v_weights, (0, NVP - N_V))
    ev0 = jnp.pad(edge_verts[:, 0].astype(jnp.int32), (0, NEP - N_E))
    ev1 = jnp.pad(edge_verts[:, 1].astype(jnp.int32), (0, NEP - N_E))
    ewp = jnp.pad(edge_weights, (0, NEP - N_E))
    tv0 = jnp.pad(tri_verts[:, 0].astype(jnp.int32), (0, NTP - N_T))
    tv1 = jnp.pad(tri_verts[:, 1].astype(jnp.int32), (0, NTP - N_T))
    tv2 = jnp.pad(tri_verts[:, 2].astype(jnp.int32), (0, NTP - N_T))
    twp = jnp.pad(tri_weights, (0, NTP - N_T))

    maxsq = pl.pallas_call(
        _maxsq_body,
        grid=(NGRID,),
        in_specs=[pl.BlockSpec((1, VBLK), lambda i: (0, i))] * 3,
        out_specs=pl.BlockSpec(memory_space=pltpu.SMEM),
        out_shape=jax.ShapeDtypeStruct((1, 1), jnp.float32),
    )(cx, cy, cz)

    tbl = pl.pallas_call(
        _quant_body,
        grid=(NGRID,),
        in_specs=[
            pl.BlockSpec(memory_space=pltpu.SMEM),
            pl.BlockSpec((1, VBLK), lambda i: (0, i)),
            pl.BlockSpec((1, VBLK), lambda i: (0, i)),
            pl.BlockSpec((1, VBLK), lambda i: (0, i)),
            pl.BlockSpec((D, 3), lambda i: (0, 0)),
        ],
        out_specs=pl.BlockSpec((D, VBLK), lambda i: (0, i)),
        out_shape=jax.ShapeDtypeStruct((D, NVP), jnp.int32),
    )(maxsq, cx, cy, cz, dirs)

    hists = _sc_hist(tbl.reshape(-1), vwp, ev0, ev1, ewp,
                     tv0, tv1, tv2, twp)

    out = pl.pallas_call(
        _fin_body,
        in_specs=[pl.BlockSpec((NW * NSUB, H), lambda: (0, 0))],
        out_specs=pl.BlockSpec((D, H), lambda: (0, 0)),
        out_shape=jax.ShapeDtypeStruct((D, H), jnp.float32),
    )(hists.reshape(NW * NSUB, H))

    return out


# speedup vs baseline: 1.0637x; 1.0637x over previous
import functools

import jax
import jax.numpy as jnp
from jax import lax
from jax.experimental import pallas as pl
from jax.experimental.pallas import tpu as pltpu
from jax.experimental.pallas import tpu_sc as plsc

D = 32
H = 256
NSUB = 16
SHW = NSUB * H

N_V = 100000
N_E = 500000
N_T = 500000

NW = 32
NC = 2

NVP = 102400
NEP = 524288
NTP = 524288

SB = 1024
NSV = NVP // SB
NSE = NEP // SB
NST = NTP // SB

VBLK = 2048
NGRID = NVP // VBLK


def _maxsq_body(x_ref, y_ref, z_ref, o_ref):
    i = pl.program_id(0)
    x = x_ref[...]
    y = y_ref[...]
    z = z_ref[...]
    m = jnp.max(x * x + y * y + z * z)

    @pl.when(i == 0)
    def _():
        o_ref[0, 0] = m

    @pl.when(i > 0)
    def _():
        o_ref[0, 0] = jnp.maximum(o_ref[0, 0], m)


def _quant_body(ms_ref, x_ref, y_ref, z_ref, d_ref, o_ref):
    m = jnp.sqrt(ms_ref[0, 0])
    dm = d_ref[...]
    h = (dm[:, 0:1] * x_ref[...]
         + dm[:, 1:2] * y_ref[...]
         + dm[:, 2:3] * z_ref[...])
    idx = jnp.ceil((jnp.float32(H - 1) * (m + h)) / (jnp.float32(2.0) * m))
    idx = jnp.clip(idx, 0.0, jnp.float32(H - 1))
    o_ref[...] = idx.astype(jnp.int32) * NSUB


def _fin_body(h_ref, o_ref):
    acc = h_ref[...]
    s_i = lax.broadcasted_iota(jnp.int32, (H, H), 0)
    t_i = lax.broadcasted_iota(jnp.int32, (H, H), 1)
    tri = (s_i <= t_i).astype(jnp.float32)
    o_ref[...] = jnp.dot(acc, tri, preferred_element_type=jnp.float32)


def _sc_hist_body(tbl, vw, ev0, ev1, ew, tv0, tv1, tv2, tw, out,
                  binv, subh, hist, ids, wbuf, sem):
    wid = lax.axis_index("s") * NC + lax.axis_index("c")

    zeros16 = jnp.zeros((16,), jnp.float32)

    def zi(i, _):
        subh[pl.ds(i * 16, 16)] = zeros16
        return 0

    lax.fori_loop(0, SHW // 16, zi, 0)

    pltpu.sync_copy(tbl.at[pl.ds(wid * NVP, NVP)], binv)

    laneoff = lax.iota(jnp.int32, 16)

    def vsb(k, carry):
        pltpu.sync_copy(vw.at[pl.ds(k * SB, SB)], wbuf.at[0])

        def vbody(g, c2):
            b16 = binv[pl.ds(k * SB + g * 16, 16)]
            w16 = wbuf[0, pl.ds(g * 16, 16)]
            plsc.addupdate_scatter(subh, [b16 + laneoff], w16)
            return c2

        return lax.fori_loop(0, SB // 16, vbody, carry, unroll=8)

    lax.fori_loop(0, NSV, vsb, 0)

    def simplex_pass(cols, w_hbm, n_super, negate):
        rows_per = len(cols)

        def stage(k, p):
            for r in range(rows_per):
                pltpu.async_copy(cols[r].at[pl.ds(k * SB, SB)],
                                 ids.at[p, pl.ds(r * SB, SB)],
                                 sem.at[p])
            pltpu.async_copy(w_hbm.at[pl.ds(k * SB, SB)], wbuf.at[p],
                             sem.at[p])

        def wait(p):
            pltpu.make_async_copy(
                cols[0].at[pl.ds(0, rows_per * SB)],
                ids.at[p, pl.ds(0, rows_per * SB)],
                sem.at[p]).wait()
            pltpu.make_async_copy(
                w_hbm.at[pl.ds(0, SB)], wbuf.at[p], sem.at[p]).wait()

        def compute(p):
            def body(g, c2):
                i0 = ids[p, pl.ds(g * 16, 16)]
                i1 = ids[p, pl.ds(SB + g * 16, 16)]
                b = jnp.maximum(plsc.load_gather(binv, [i0]),
                                plsc.load_gather(binv, [i1]))
                if rows_per == 3:
                    i2 = ids[p, pl.ds(2 * SB + g * 16, 16)]
                    b = jnp.maximum(b, plsc.load_gather(binv, [i2]))
                w16 = wbuf[p, pl.ds(g * 16, 16)]
                if negate:
                    w16 = -w16
                plsc.addupdate_scatter(subh, [b + laneoff], w16)
                return c2

            lax.fori_loop(0, SB // 16, body, 0, unroll=8)

        stage(0, 0)

        def pair(j, carry):
            stage(2 * j + 1, 1)
            wait(0)
            compute(0)

            @pl.when(j + 1 < n_super // 2)
            def _():
                stage(2 * j + 2, 0)

            wait(1)
            compute(1)
            return carry

        lax.fori_loop(0, n_super // 2, pair, 0)

    simplex_pass([ev0, ev1], ew, NSE, False)
    simplex_pass([tv0, tv1, tv2], tw, NST, True)

    def red(g, _):
        bi = lax.iota(jnp.int32, 16) * NSUB + g * (16 * NSUB)

        def addj(j, a):
            return a + plsc.load_gather(subh, [bi + j])

        hist[pl.ds(g * 16, 16)] = lax.fori_loop(
            0, NSUB, addj, jnp.zeros((16,), jnp.float32), unroll=NSUB)
        return 0

    lax.fori_loop(0, H // 16, red, 0)
    pltpu.sync_copy(hist, out.at[pl.ds(wid * H, H)])


def _make_sc_hist():
    mesh = plsc.VectorSubcoreMesh(core_axis_name="c", subcore_axis_name="s")
    return functools.partial(
        pl.kernel,
        mesh=mesh,
        compiler_params=pltpu.CompilerParams(
            needs_layout_passes=False, use_tc_tiling_on_sc=False),
        out_type=jax.ShapeDtypeStruct((NW * H,), jnp.float32),
        scratch_types=[
            pltpu.VMEM((NVP,), jnp.int32),
            pltpu.VMEM((SHW,), jnp.float32),
            pltpu.VMEM((H,), jnp.float32),
            pltpu.VMEM((2, 3 * SB), jnp.int32),
            pltpu.VMEM((2, SB), jnp.float32),
            pltpu.SemaphoreType.DMA((2,)),
        ],
    )(_sc_hist_body)


_sc_hist = _make_sc_hist()


def kernel(v_coords, v_weights, edge_verts, edge_weights, tri_verts,
           tri_weights, dirs):
    cx = jnp.pad(v_coords[:, 0], (0, NVP - N_V)).reshape(1, NVP)
    cy = jnp.pad(v_coords[:, 1], (0, NVP - N_V)).reshape(1, NVP)
    cz = jnp.pad(v_coords[:, 2], (0, NVP - N_V)).reshape(1, NVP)
    vwp = jnp.pad(v_weights, (0, NVP - N_V))
    ev0 = jnp.pad(edge_verts[:, 0].astype(jnp.int32), (0, NEP - N_E))
    ev1 = jnp.pad(edge_verts[:, 1].astype(jnp.int32), (0, NEP - N_E))
    ewp = jnp.pad(edge_weights, (0, NEP - N_E))
    tv0 = jnp.pad(tri_verts[:, 0].astype(jnp.int32), (0, NTP - N_T))
    tv1 = jnp.pad(tri_verts[:, 1].astype(jnp.int32), (0, NTP - N_T))
    tv2 = jnp.pad(tri_verts[:, 2].astype(jnp.int32), (0, NTP - N_T))
    twp = jnp.pad(tri_weights, (0, NTP - N_T))

    maxsq = pl.pallas_call(
        _maxsq_body,
        grid=(NGRID,),
        in_specs=[pl.BlockSpec((1, VBLK), lambda i: (0, i))] * 3,
        out_specs=pl.BlockSpec(memory_space=pltpu.SMEM),
        out_shape=jax.ShapeDtypeStruct((1, 1), jnp.float32),
    )(cx, cy, cz)

    tbl = pl.pallas_call(
        _quant_body,
        grid=(NGRID,),
        in_specs=[
            pl.BlockSpec(memory_space=pltpu.SMEM),
            pl.BlockSpec((1, VBLK), lambda i: (0, i)),
            pl.BlockSpec((1, VBLK), lambda i: (0, i)),
            pl.BlockSpec((1, VBLK), lambda i: (0, i)),
            pl.BlockSpec((D, 3), lambda i: (0, 0)),
        ],
        out_specs=pl.BlockSpec((D, VBLK), lambda i: (0, i)),
        out_shape=jax.ShapeDtypeStruct((D, NVP), jnp.int32),
    )(maxsq, cx, cy, cz, dirs)

    hists = _sc_hist(tbl.reshape(-1), vwp, ev0, ev1, ewp,
                     tv0, tv1, tv2, twp)

    out = pl.pallas_call(
        _fin_body,
        in_specs=[pl.BlockSpec((D, H), lambda: (0, 0))],
        out_specs=pl.BlockSpec((D, H), lambda: (0, 0)),
        out_shape=jax.ShapeDtypeStruct((D, H), jnp.float32),
    )(hists.reshape(D, H))

    return out


# speedup vs baseline: 1.0987x; 1.0329x over previous
import functools

import jax
import jax.numpy as jnp
from jax import lax
from jax.experimental import pallas as pl
from jax.experimental.pallas import tpu as pltpu
from jax.experimental.pallas import tpu_sc as plsc

D = 32
H = 256
NSUB = 16
SHW = NSUB * H

N_V = 100000
N_E = 500000
N_T = 500000

NW = 32
NC = 2

NVP = 102400
NEP = 501760
NTP = 501760

SB = 1024
NSV = NVP // SB
NSE = NEP // SB
NST = NTP // SB

VBLK = 2048
NGRID = NVP // VBLK


def _maxsq_body(x_ref, y_ref, z_ref, o_ref):
    i = pl.program_id(0)
    x = x_ref[...]
    y = y_ref[...]
    z = z_ref[...]
    m = jnp.max(x * x + y * y + z * z)

    @pl.when(i == 0)
    def _():
        o_ref[0, 0] = m

    @pl.when(i > 0)
    def _():
        o_ref[0, 0] = jnp.maximum(o_ref[0, 0], m)


def _quant_body(ms_ref, x_ref, y_ref, z_ref, d_ref, o_ref):
    m = jnp.sqrt(ms_ref[0, 0])
    dm = d_ref[...]
    h = (dm[:, 0:1] * x_ref[...]
         + dm[:, 1:2] * y_ref[...]
         + dm[:, 2:3] * z_ref[...])
    idx = jnp.ceil((jnp.float32(H - 1) * (m + h)) / (jnp.float32(2.0) * m))
    idx = jnp.clip(idx, 0.0, jnp.float32(H - 1))
    o_ref[...] = idx.astype(jnp.int32) * NSUB


def _fin_body(h_ref, o_ref):
    acc = h_ref[...]
    s_i = lax.broadcasted_iota(jnp.int32, (H, H), 0)
    t_i = lax.broadcasted_iota(jnp.int32, (H, H), 1)
    tri = (s_i <= t_i).astype(jnp.float32)
    o_ref[...] = jnp.dot(acc, tri, preferred_element_type=jnp.float32)


def _sc_hist_body(tbl, vw, ev0, ev1, ew, tv0, tv1, tv2, tw, out,
                  binv, subh, hist, ids, wbuf, sem):
    wid = lax.axis_index("s") * NC + lax.axis_index("c")

    zeros16 = jnp.zeros((16,), jnp.float32)

    def zi(i, _):
        subh[pl.ds(i * 16, 16)] = zeros16
        return 0

    lax.fori_loop(0, SHW // 16, zi, 0)

    pltpu.sync_copy(tbl.at[pl.ds(wid * NVP, NVP)], binv)

    laneoff = lax.iota(jnp.int32, 16)

    def vsb(k, carry):
        pltpu.sync_copy(vw.at[pl.ds(k * SB, SB)], wbuf.at[0])

        def vbody(g, c2):
            b16 = binv[pl.ds(k * SB + g * 16, 16)]
            w16 = wbuf[0, pl.ds(g * 16, 16)]
            plsc.addupdate_scatter(subh, [b16 + laneoff], w16)
            return c2

        return lax.fori_loop(0, SB // 16, vbody, carry, unroll=8)

    lax.fori_loop(0, NSV, vsb, 0)

    def simplex_pass(cols, w_hbm, n_super, negate):
        rows_per = len(cols)

        def stage(k, p):
            for r in range(rows_per):
                pltpu.async_copy(cols[r].at[pl.ds(k * SB, SB)],
                                 ids.at[p, pl.ds(r * SB, SB)],
                                 sem.at[p])
            pltpu.async_copy(w_hbm.at[pl.ds(k * SB, SB)], wbuf.at[p],
                             sem.at[p])

        def wait(p):
            pltpu.make_async_copy(
                cols[0].at[pl.ds(0, rows_per * SB)],
                ids.at[p, pl.ds(0, rows_per * SB)],
                sem.at[p]).wait()
            pltpu.make_async_copy(
                w_hbm.at[pl.ds(0, SB)], wbuf.at[p], sem.at[p]).wait()

        def compute(p):
            def body(g, c2):
                i0 = ids[p, pl.ds(g * 16, 16)]
                i1 = ids[p, pl.ds(SB + g * 16, 16)]
                b = jnp.maximum(plsc.load_gather(binv, [i0]),
                                plsc.load_gather(binv, [i1]))
                if rows_per == 3:
                    i2 = ids[p, pl.ds(2 * SB + g * 16, 16)]
                    b = jnp.maximum(b, plsc.load_gather(binv, [i2]))
                w16 = wbuf[p, pl.ds(g * 16, 16)]
                if negate:
                    w16 = -w16
                plsc.addupdate_scatter(subh, [b + laneoff], w16)
                return c2

            lax.fori_loop(0, SB // 16, body, 0, unroll=8)

        stage(0, 0)

        def pair(j, carry):
            stage(2 * j + 1, 1)
            wait(0)
            compute(0)

            @pl.when(j + 1 < n_super // 2)
            def _():
                stage(2 * j + 2, 0)

            wait(1)
            compute(1)
            return carry

        lax.fori_loop(0, n_super // 2, pair, 0)

    simplex_pass([ev0, ev1], ew, NSE, False)
    simplex_pass([tv0, tv1, tv2], tw, NST, True)

    def red(g, _):
        bi = lax.iota(jnp.int32, 16) * NSUB + g * (16 * NSUB)

        def addj(j, a):
            return a + plsc.load_gather(subh, [bi + j])

        hist[pl.ds(g * 16, 16)] = lax.fori_loop(
            0, NSUB, addj, jnp.zeros((16,), jnp.float32), unroll=NSUB)
        return 0

    lax.fori_loop(0, H // 16, red, 0)
    pltpu.sync_copy(hist, out.at[pl.ds(wid * H, H)])


def _make_sc_hist():
    mesh = plsc.VectorSubcoreMesh(core_axis_name="c", subcore_axis_name="s")
    return functools.partial(
        pl.kernel,
        mesh=mesh,
        compiler_params=pltpu.CompilerParams(
            needs_layout_passes=False, use_tc_tiling_on_sc=False),
        out_type=jax.ShapeDtypeStruct((NW * H,), jnp.float32),
        scratch_types=[
            pltpu.VMEM((NVP,), jnp.int32),
            pltpu.VMEM((SHW,), jnp.float32),
            pltpu.VMEM((H,), jnp.float32),
            pltpu.VMEM((2, 3 * SB), jnp.int32),
            pltpu.VMEM((2, SB), jnp.float32),
            pltpu.SemaphoreType.DMA((2,)),
        ],
    )(_sc_hist_body)


_sc_hist = _make_sc_hist()


def kernel(v_coords, v_weights, edge_verts, edge_weights, tri_verts,
           tri_weights, dirs):
    cx = jnp.pad(v_coords[:, 0], (0, NVP - N_V)).reshape(1, NVP)
    cy = jnp.pad(v_coords[:, 1], (0, NVP - N_V)).reshape(1, NVP)
    cz = jnp.pad(v_coords[:, 2], (0, NVP - N_V)).reshape(1, NVP)
    vwp = jnp.pad(v_weights, (0, NVP - N_V))
    ev0 = jnp.pad(edge_verts[:, 0].astype(jnp.int32), (0, NEP - N_E))
    ev1 = jnp.pad(edge_verts[:, 1].astype(jnp.int32), (0, NEP - N_E))
    ewp = jnp.pad(edge_weights, (0, NEP - N_E))
    tv0 = jnp.pad(tri_verts[:, 0].astype(jnp.int32), (0, NTP - N_T))
    tv1 = jnp.pad(tri_verts[:, 1].astype(jnp.int32), (0, NTP - N_T))
    tv2 = jnp.pad(tri_verts[:, 2].astype(jnp.int32), (0, NTP - N_T))
    twp = jnp.pad(tri_weights, (0, NTP - N_T))

    maxsq = pl.pallas_call(
        _maxsq_body,
        grid=(NGRID,),
        in_specs=[pl.BlockSpec((1, VBLK), lambda i: (0, i))] * 3,
        out_specs=pl.BlockSpec(memory_space=pltpu.SMEM),
        out_shape=jax.ShapeDtypeStruct((1, 1), jnp.float32),
    )(cx, cy, cz)

    tbl = pl.pallas_call(
        _quant_body,
        grid=(NGRID,),
        in_specs=[
            pl.BlockSpec(memory_space=pltpu.SMEM),
            pl.BlockSpec((1, VBLK), lambda i: (0, i)),
            pl.BlockSpec((1, VBLK), lambda i: (0, i)),
            pl.BlockSpec((1, VBLK), lambda i: (0, i)),
            pl.BlockSpec((D, 3), lambda i: (0, 0)),
        ],
        out_specs=pl.BlockSpec((D, VBLK), lambda i: (0, i)),
        out_shape=jax.ShapeDtypeStruct((D, NVP), jnp.int32),
    )(maxsq, cx, cy, cz, dirs)

    hists = _sc_hist(tbl.reshape(-1), vwp, ev0, ev1, ewp,
                     tv0, tv1, tv2, twp)

    out = pl.pallas_call(
        _fin_body,
        in_specs=[pl.BlockSpec((D, H), lambda: (0, 0))],
        out_specs=pl.BlockSpec((D, H), lambda: (0, 0)),
        out_shape=jax.ShapeDtypeStruct((D, H), jnp.float32),
    )(hists.reshape(D, H))

    return out


# speedup vs baseline: 1.1015x; 1.0025x over previous
import functools

import jax
import jax.numpy as jnp
from jax import lax
from jax.experimental import pallas as pl
from jax.experimental.pallas import tpu as pltpu
from jax.experimental.pallas import tpu_sc as plsc

D = 32
H = 256
NSUB = 16
SHW = NSUB * H

N_V = 100000
N_E = 500000
N_T = 500000

NW = 32
NC = 2

NVP = 102400
NEP = 501760
NTP = 501760

SB = 1024
NSV = NVP // SB
NSE = NEP // SB
NST = NTP // SB

VBLK = 2048
NGRID = NVP // VBLK


def _maxsq_body(x_ref, y_ref, z_ref, o_ref):
    i = pl.program_id(0)
    x = x_ref[...]
    y = y_ref[...]
    z = z_ref[...]
    m = jnp.max(x * x + y * y + z * z)

    @pl.when(i == 0)
    def _():
        o_ref[0, 0] = m

    @pl.when(i > 0)
    def _():
        o_ref[0, 0] = jnp.maximum(o_ref[0, 0], m)


def _quant_body(ms_ref, x_ref, y_ref, z_ref, d_ref, o_ref):
    m = jnp.sqrt(ms_ref[0, 0])
    dm = d_ref[...]
    h = (dm[:, 0:1] * x_ref[...]
         + dm[:, 1:2] * y_ref[...]
         + dm[:, 2:3] * z_ref[...])
    idx = jnp.ceil((jnp.float32(H - 1) * (m + h)) / (jnp.float32(2.0) * m))
    idx = jnp.clip(idx, 0.0, jnp.float32(H - 1))
    o_ref[...] = idx.astype(jnp.int32) * NSUB


def _fin_body(h_ref, o_ref):
    acc = h_ref[...]
    s_i = lax.broadcasted_iota(jnp.int32, (H, H), 0)
    t_i = lax.broadcasted_iota(jnp.int32, (H, H), 1)
    tri = (s_i <= t_i).astype(jnp.float32)
    o_ref[...] = jnp.dot(acc, tri, preferred_element_type=jnp.float32)


def _sc_hist_body(tbl, vw, ev0, ev1, ew, tv0, tv1, tv2, tw, out,
                  binv, subh, hist, ids, wbuf, sem):
    wid = lax.axis_index("s") * NC + lax.axis_index("c")

    zeros16 = jnp.zeros((16,), jnp.float32)

    def zi(i, _):
        subh[pl.ds(i * 16, 16)] = zeros16
        return 0

    lax.fori_loop(0, SHW // 16, zi, 0)

    pltpu.sync_copy(tbl.at[pl.ds(wid * NVP, NVP)], binv)

    laneoff = lax.iota(jnp.int32, 16)

    def vsb(k, carry):
        pltpu.sync_copy(vw.at[pl.ds(k * SB, SB)], wbuf.at[0])

        def vbody(g, c2):
            b16 = binv[pl.ds(k * SB + g * 16, 16)]
            w16 = wbuf[0, pl.ds(g * 16, 16)]
            plsc.addupdate_scatter(subh, [b16 + laneoff], w16)
            return c2

        return lax.fori_loop(0, SB // 16, vbody, carry, unroll=16)

    lax.fori_loop(0, NSV, vsb, 0)

    def simplex_pass(cols, w_hbm, n_super, negate):
        rows_per = len(cols)

        def stage(k, p):
            for r in range(rows_per):
                pltpu.async_copy(cols[r].at[pl.ds(k * SB, SB)],
                                 ids.at[p, pl.ds(r * SB, SB)],
                                 sem.at[p])
            pltpu.async_copy(w_hbm.at[pl.ds(k * SB, SB)], wbuf.at[p],
                             sem.at[p])

        def wait(p):
            pltpu.make_async_copy(
                cols[0].at[pl.ds(0, rows_per * SB)],
                ids.at[p, pl.ds(0, rows_per * SB)],
                sem.at[p]).wait()
            pltpu.make_async_copy(
                w_hbm.at[pl.ds(0, SB)], wbuf.at[p], sem.at[p]).wait()

        def compute(p):
            def body(g, c2):
                i0 = ids[p, pl.ds(g * 16, 16)]
                i1 = ids[p, pl.ds(SB + g * 16, 16)]
                b = jnp.maximum(plsc.load_gather(binv, [i0]),
                                plsc.load_gather(binv, [i1]))
                if rows_per == 3:
                    i2 = ids[p, pl.ds(2 * SB + g * 16, 16)]
                    b = jnp.maximum(b, plsc.load_gather(binv, [i2]))
                w16 = wbuf[p, pl.ds(g * 16, 16)]
                if negate:
                    w16 = -w16
                plsc.addupdate_scatter(subh, [b + laneoff], w16)
                return c2

            lax.fori_loop(0, SB // 16, body, 0, unroll=16)

        stage(0, 0)

        def pair(j, carry):
            stage(2 * j + 1, 1)
            wait(0)
            compute(0)

            @pl.when(j + 1 < n_super // 2)
            def _():
                stage(2 * j + 2, 0)

            wait(1)
            compute(1)
            return carry

        lax.fori_loop(0, n_super // 2, pair, 0)

    simplex_pass([ev0, ev1], ew, NSE, False)
    simplex_pass([tv0, tv1, tv2], tw, NST, True)

    def red(g, _):
        bi = lax.iota(jnp.int32, 16) * NSUB + g * (16 * NSUB)

        def addj(j, a):
            return a + plsc.load_gather(subh, [bi + j])

        hist[pl.ds(g * 16, 16)] = lax.fori_loop(
            0, NSUB, addj, jnp.zeros((16,), jnp.float32), unroll=NSUB)
        return 0

    lax.fori_loop(0, H // 16, red, 0)
    pltpu.sync_copy(hist, out.at[pl.ds(wid * H, H)])


def _make_sc_hist():
    mesh = plsc.VectorSubcoreMesh(core_axis_name="c", subcore_axis_name="s")
    return functools.partial(
        pl.kernel,
        mesh=mesh,
        compiler_params=pltpu.CompilerParams(
            needs_layout_passes=False, use_tc_tiling_on_sc=False),
        out_type=jax.ShapeDtypeStruct((NW * H,), jnp.float32),
        scratch_types=[
            pltpu.VMEM((NVP,), jnp.int32),
            pltpu.VMEM((SHW,), jnp.float32),
            pltpu.VMEM((H,), jnp.float32),
            pltpu.VMEM((2, 3 * SB), jnp.int32),
            pltpu.VMEM((2, SB), jnp.float32),
            pltpu.SemaphoreType.DMA((2,)),
        ],
    )(_sc_hist_body)


_sc_hist = _make_sc_hist()


def kernel(v_coords, v_weights, edge_verts, edge_weights, tri_verts,
           tri_weights, dirs):
    cx = jnp.pad(v_coords[:, 0], (0, NVP - N_V)).reshape(1, NVP)
    cy = jnp.pad(v_coords[:, 1], (0, NVP - N_V)).reshape(1, NVP)
    cz = jnp.pad(v_coords[:, 2], (0, NVP - N_V)).reshape(1, NVP)
    vwp = jnp.pad(v_weights, (0, NVP - N_V))
    ev0 = jnp.pad(edge_verts[:, 0].astype(jnp.int32), (0, NEP - N_E))
    ev1 = jnp.pad(edge_verts[:, 1].astype(jnp.int32), (0, NEP - N_E))
    ewp = jnp.pad(edge_weights, (0, NEP - N_E))
    tv0 = jnp.pad(tri_verts[:, 0].astype(jnp.int32), (0, NTP - N_T))
    tv1 = jnp.pad(tri_verts[:, 1].astype(jnp.int32), (0, NTP - N_T))
    tv2 = jnp.pad(tri_verts[:, 2].astype(jnp.int32), (0, NTP - N_T))
    twp = jnp.pad(tri_weights, (0, NTP - N_T))

    maxsq = pl.pallas_call(
        _maxsq_body,
        grid=(NGRID,),
        in_specs=[pl.BlockSpec((1, VBLK), lambda i: (0, i))] * 3,
        out_specs=pl.BlockSpec(memory_space=pltpu.SMEM),
        out_shape=jax.ShapeDtypeStruct((1, 1), jnp.float32),
    )(cx, cy, cz)

    tbl = pl.pallas_call(
        _quant_body,
        grid=(NGRID,),
        in_specs=[
            pl.BlockSpec(memory_space=pltpu.SMEM),
            pl.BlockSpec((1, VBLK), lambda i: (0, i)),
            pl.BlockSpec((1, VBLK), lambda i: (0, i)),
            pl.BlockSpec((1, VBLK), lambda i: (0, i)),
            pl.BlockSpec((D, 3), lambda i: (0, 0)),
        ],
        out_specs=pl.BlockSpec((D, VBLK), lambda i: (0, i)),
        out_shape=jax.ShapeDtypeStruct((D, NVP), jnp.int32),
    )(maxsq, cx, cy, cz, dirs)

    hists = _sc_hist(tbl.reshape(-1), vwp, ev0, ev1, ewp,
                     tv0, tv1, tv2, twp)

    out = pl.pallas_call(
        _fin_body,
        in_specs=[pl.BlockSpec((D, H), lambda: (0, 0))],
        out_specs=pl.BlockSpec((D, H), lambda: (0, 0)),
        out_shape=jax.ShapeDtypeStruct((D, H), jnp.float32),
    )(hists.reshape(D, H))

    return out
